# baseline (speedup 1.0000x reference)
import functools

import jax
import jax.numpy as jnp
from jax import lax
from jax.experimental import pallas as pl
from jax.experimental.pallas import tpu as pltpu
from jax.experimental.pallas import tpu_sc as plsc

RANGE = 15
GRANU = 16
L = RANGE * GRANU
N_TOTAL = 16 * 256 * 56 * 56
NC, NS = 2, 16
NW = NC * NS
PER_W = N_TOTAL // NW
CHUNK = 14336
NB = 4
NCH = PER_W // CHUNK
ROUNDS = NCH // NB
TAIL = NCH - ROUNDS * NB
VPC = CHUNK // 16


def _take(v, idx):
    return v.at[idx].get(mode="promise_in_bounds")


def _tec_body(x_hbm, tab_hbm, slog_hbm, out_hbm,
              tab_v, d_v, misc_v, in0, in1, in2, in3, out0, out1, out2, out3,
              si0, si1, si2, si3, so0, so1, so2, so3):
    cid = lax.axis_index("c")
    sid = lax.axis_index("s")
    wid = sid * NC + cid
    base = wid * PER_W

    pltpu.sync_copy(slog_hbm, misc_v)
    pltpu.sync_copy(tab_hbm, tab_v)

    slog = misc_v[...]
    scale = jnp.exp(slog)
    inv240 = 240.0 / scale
    sc15 = scale / float(RANGE)

    zero = jnp.zeros((16,), jnp.float32)
    for r in range(16):
        d_v[pl.ds(16 * r, 16)] = zero
    iota = lax.iota(jnp.int32, 16)
    lane15 = jnp.full((16,), 15, jnp.int32)
    carry = jnp.zeros((16,), jnp.float32)
    for r in range(RANGE):
        v = tab_v[r]
        e = jnp.exp(v)
        s = e
        for k in (1, 2, 4, 8):
            s = s + _take(s, iota ^ k)
        p = e / s
        csum = p
        for k in (1, 2, 4, 8):
            shifted = _take(csum, jnp.maximum(iota - k, 0))
            csum = csum + jnp.where(iota >= k, shifted, 0.0)
        plsc.store_scatter(d_v, [iota + (16 * r + 1)], (carry + csum) * sc15)
        carry = carry + _take(csum, lane15)

    ins = (in0, in1, in2, in3)
    outs = (out0, out1, out2, out3)
    isems = (si0, si1, si2, si3)
    osems = (so0, so1, so2, so3)
    half = jnp.full((16,), 0.5, jnp.float32)

    def in_copy(g, b):
        return pltpu.make_async_copy(
            x_hbm.at[pl.ds(base + g * CHUNK, CHUNK)], ins[b], isems[b])

    def out_copy(g, b):
        return pltpu.make_async_copy(
            outs[b], out_hbm.at[pl.ds(base + g * CHUNK, CHUNK)], osems[b])

    for b in range(NB):
        in_copy(b, b).start()

    def compute(b):
        inb = ins[b]
        outb = outs[b]

        @plsc.parallel_loop(0, VPC, step=1, unroll=8)
        def _(i):
            off = i * 16
            xv = inb[pl.ds(off, 16)]
            u = xv * inv240 + half
            u = jnp.minimum(u, 240.5)
            u = jnp.maximum(u, 0.0)
            ji = u.astype(jnp.int32)
            outb[pl.ds(off, 16)] = plsc.load_gather(d_v, [ji])

    def outer(k, _):
        for b in range(NB):
            g = NB * k + b
            in_copy(g, b).wait()

            @pl.when(k > 0)
            def _():
                out_copy(g - NB, b).wait()

            compute(b)
            out_copy(g, b).start()

            if b < TAIL:
                in_copy(g + NB, b).start()
            else:
                @pl.when(k < ROUNDS - 1)
                def _():
                    in_copy(g + NB, b).start()

        return 0

    lax.fori_loop(0, ROUNDS, outer, 0)
    for b in range(TAIL):
        g = NB * ROUNDS + b
        in_copy(g, b).wait()
        out_copy(g - NB, b).wait()
        compute(b)
        out_copy(g, b).start()
    for i in range(NB):
        g = NCH - NB + i
        out_copy(g, g % NB).wait()


@jax.jit
def kernel(x, table, scale_log):
    mesh = plsc.VectorSubcoreMesh(core_axis_name="c", subcore_axis_name="s")
    k = pl.kernel(
        _tec_body,
        out_type=jax.ShapeDtypeStruct((N_TOTAL,), jnp.float32),
        mesh=mesh,
        compiler_params=pltpu.CompilerParams(needs_layout_passes=False),
        scratch_types=[
            pltpu.VMEM((RANGE, GRANU), jnp.float32),
            pltpu.VMEM((256,), jnp.float32),
            pltpu.VMEM((16,), jnp.float32),
        ] + [pltpu.VMEM((CHUNK,), jnp.float32)] * (2 * NB)
          + [pltpu.SemaphoreType.DMA] * (2 * NB),
    )
    slog16 = jnp.full((16,), scale_log, jnp.float32)
    x6 = x.reshape(16, 2, 128, 56, 7, 8)
    xp = x6.transpose(0, 3, 4, 1, 5, 2).reshape(-1)
    out = k(xp, table, slog16)
    o6 = out.reshape(16, 56, 7, 2, 8, 128)
    return o6.transpose(0, 3, 5, 1, 2, 4).reshape(x.shape)


# speedup vs baseline: 1.0448x; 1.0448x over previous
import jax
import jax.numpy as jnp
from jax import lax
from jax.experimental import pallas as pl
from jax.experimental.pallas import tpu as pltpu
from jax.experimental.pallas import tpu_sc as plsc

RANGE = 15
GRANU = 16
L = RANGE * GRANU
N_TOTAL = 16 * 256 * 56 * 56
NC, NS = 2, 16
NW = NC * NS
PER_W = N_TOTAL // NW
CHUNK = 8192
NB = 3
NCH = PER_W // CHUNK
ROUNDS = NCH // NB
TAIL = NCH - ROUNDS * NB
VPC = CHUNK // 16


def _take(v, idx):
    return v.at[idx].get(mode="promise_in_bounds")


def _tec_body(x_hbm, tab_hbm, slog_hbm, out_hbm,
              tab_v, d_v, misc_v, in0, in1, in2, out0, out1, out2,
              si0, si1, si2, so0, so1, so2):
    cid = lax.axis_index("c")
    sid = lax.axis_index("s")
    wid = sid * NC + cid
    base = wid * PER_W

    ins = (in0, in1, in2)
    outs = (out0, out1, out2)
    isems = (si0, si1, si2)
    osems = (so0, so1, so2)

    def in_copy(g, b):
        return pltpu.make_async_copy(
            x_hbm.at[pl.ds(base + g * CHUNK, CHUNK)], ins[b], isems[b])

    def out_copy(g, b):
        return pltpu.make_async_copy(
            outs[b], out_hbm.at[pl.ds(base + g * CHUNK, CHUNK)], osems[b])

    for b in range(NB):
        in_copy(b, b).start()

    pltpu.sync_copy(slog_hbm, misc_v)
    pltpu.sync_copy(tab_hbm, tab_v)

    slog = misc_v[...]
    scale = jnp.exp(slog)
    inv240 = 240.0 / scale
    sc15 = scale / float(RANGE)

    zero = jnp.zeros((16,), jnp.float32)
    for r in range(16):
        d_v[pl.ds(16 * r, 16)] = zero
    iota = lax.iota(jnp.int32, 16)
    lane15 = jnp.full((16,), 15, jnp.int32)
    carry = jnp.zeros((16,), jnp.float32)
    for r in range(RANGE):
        v = tab_v[r]
        e = jnp.exp(v)
        s = e
        for k in (1, 2, 4, 8):
            s = s + _take(s, iota ^ k)
        p = e / s
        csum = p
        for k in (1, 2, 4, 8):
            shifted = _take(csum, jnp.maximum(iota - k, 0))
            csum = csum + jnp.where(iota >= k, shifted, 0.0)
        plsc.store_scatter(d_v, [iota + (16 * r + 1)], (carry + csum) * sc15)
        carry = carry + _take(csum, lane15)

    magic = jnp.full((16,), 8388608.0, jnp.float32)

    def compute(b):
        inb = ins[b]
        outb = outs[b]

        @plsc.parallel_loop(0, VPC, step=1, unroll=8)
        def _(i):
            off = i * 16
            xv = inb[pl.ds(off, 16)]
            u = xv * inv240
            u = jnp.minimum(u, 240.0)
            u = jnp.maximum(u, 0.0)
            ji = plsc.bitcast(u + magic, jnp.int32) - 0x4B000000
            outb[pl.ds(off, 16)] = plsc.load_gather(d_v, [ji])

    def outer(k, _):
        for b in range(NB):
            g = NB * k + b
            in_copy(g, b).wait()

            @pl.when(k > 0)
            def _():
                out_copy(g - NB, b).wait()

            compute(b)
            out_copy(g, b).start()

            if b < TAIL:
                in_copy(g + NB, b).start()
            else:
                @pl.when(k < ROUNDS - 1)
                def _():
                    in_copy(g + NB, b).start()

        return 0

    lax.fori_loop(0, ROUNDS, outer, 0)
    for b in range(TAIL):
        g = NB * ROUNDS + b
        in_copy(g, b).wait()
        out_copy(g - NB, b).wait()
        compute(b)
        out_copy(g, b).start()
    for i in range(NB):
        g = NCH - NB + i
        out_copy(g, g % NB).wait()


@jax.jit
def kernel(x, table, scale_log):
    mesh = plsc.VectorSubcoreMesh(core_axis_name="c", subcore_axis_name="s")
    k = pl.kernel(
        _tec_body,
        out_type=jax.ShapeDtypeStruct((N_TOTAL,), jnp.float32),
        mesh=mesh,
        compiler_params=pltpu.CompilerParams(needs_layout_passes=False),
        scratch_types=[
            pltpu.VMEM((RANGE, GRANU), jnp.float32),
            pltpu.VMEM((256,), jnp.float32),
            pltpu.VMEM((16,), jnp.float32),
        ] + [pltpu.VMEM((CHUNK,), jnp.float32)] * (2 * NB)
          + [pltpu.SemaphoreType.DMA] * (2 * NB),
    )
    slog16 = jnp.full((16,), scale_log, jnp.float32)
    x6 = x.reshape(16, 2, 128, 56, 7, 8)
    xp = x6.transpose(0, 3, 4, 1, 5, 2).reshape(-1)
    out = k(xp, table, slog16)
    o6 = out.reshape(16, 56, 7, 2, 8, 128)
    return o6.transpose(0, 3, 5, 1, 2, 4).reshape(x.shape)
